# baseline (speedup 1.0000x reference)
import functools

import jax
import jax.numpy as jnp
from jax import lax
from jax.experimental import pallas as pl
from jax.experimental.pallas import tpu as pltpu
from jax.experimental.pallas import tpu_sc as plsc

_EMBED_DIM = 50
_ROW = 128

_info = plsc.get_sparse_core_info()
_NC = _info.num_cores
_NS = _info.num_subcores
_NW = _NC * _NS

_CHUNK = 128
_TBLK = 16384


def _transpose_block(tt_ref, out_ref):
    t = jnp.transpose(tt_ref[...], (1, 0))
    pad = jnp.zeros((_TBLK, _ROW - _EMBED_DIM), jnp.float32)
    out_ref[...] = jnp.concatenate([t, pad], axis=1)


def _make_transpose(vocab: int):
    grid = (vocab + _TBLK - 1) // _TBLK
    return pl.pallas_call(
        _transpose_block,
        grid=(grid,),
        in_specs=[
            pl.BlockSpec((_EMBED_DIM, _TBLK), lambda i: (0, i)),
        ],
        out_specs=pl.BlockSpec((_TBLK, _ROW), lambda i: (i, 0)),
        out_shape=jax.ShapeDtypeStruct((vocab, _ROW), jnp.float32),
    )


def _make_gather(total_rows: int, vocab: int):
    rows_per_w = total_rows // _NW
    n_chunks = rows_per_w // _CHUNK
    mesh = plsc.VectorSubcoreMesh(core_axis_name="c", subcore_axis_name="s")

    @functools.partial(
        pl.kernel,
        mesh=mesh,
        out_type=jax.ShapeDtypeStruct((total_rows, _ROW), jnp.float32),
        scratch_types=[
            pltpu.VMEM((rows_per_w,), jnp.int32),
            pltpu.VMEM((2, _CHUNK, _ROW), jnp.float32),
            pltpu.SemaphoreType.DMA,
            pltpu.SemaphoreType.DMA,
            pltpu.SemaphoreType.DMA,
            pltpu.SemaphoreType.DMA,
        ],
    )
    def gather_kernel(idx_hbm, table_hbm, out_hbm, idx_v, rows_v,
                      gs0, gs1, ws0, ws1):
        wid = lax.axis_index("s") * _NC + lax.axis_index("c")
        base = wid * rows_per_w
        pltpu.sync_copy(idx_hbm.at[pl.ds(base, rows_per_w)], idx_v)

        def fire_gather(g, buf, sem):
            pltpu.async_copy(
                table_hbm.at[idx_v.at[pl.ds(g * _CHUNK, _CHUNK)]],
                rows_v.at[buf],
                sem,
            )

        def wait_gather(buf, sem):
            pltpu.make_async_copy(
                table_hbm.at[idx_v.at[pl.ds(0, _CHUNK)]], rows_v.at[buf], sem
            ).wait()

        def fire_write(g, buf, sem):
            pltpu.async_copy(
                rows_v.at[buf],
                out_hbm.at[pl.ds(base + g * _CHUNK, _CHUNK)],
                sem,
            )

        def wait_write(buf, sem):
            pltpu.make_async_copy(
                rows_v.at[buf], out_hbm.at[pl.ds(base, _CHUNK)], sem
            ).wait()

        n_pairs = n_chunks // 2
        fire_gather(0, 0, gs0)

        def body(p, carry):
            g0 = 2 * p

            @pl.when(p >= 1)
            def _():
                wait_write(1, ws1)

            fire_gather(g0 + 1, 1, gs1)
            wait_gather(0, gs0)
            fire_write(g0, 0, ws0)

            @pl.when(g0 + 2 < n_chunks)
            def _():
                wait_write(0, ws0)
                fire_gather(g0 + 2, 0, gs0)

            wait_gather(1, gs1)
            fire_write(g0 + 1, 1, ws1)
            return carry

        lax.fori_loop(0, n_pairs, body, 0)
        wait_write(0, ws0)
        wait_write(1, ws1)

    return gather_kernel


def kernel(inputs, table):
    batch, hist = inputs.shape
    total = batch * hist
    vocab = table.shape[0]
    idx = inputs.reshape(total).astype(jnp.int32)
    staged = _make_transpose(vocab)(table.T)
    out = _make_gather(total, vocab)(idx, staged)
    return out[:, :_EMBED_DIM].reshape(batch, hist, _EMBED_DIM)
